# baseline (speedup 1.0000x reference)
import functools

import jax
import jax.numpy as jnp
from jax import lax
from jax.experimental import pallas as pl
from jax.experimental.pallas import tpu as pltpu
from jax.experimental.pallas import tpu_sc as plsc

_K = 16
_R = 256
_CAP = 512
_NW = 32


def _tc_body(keys_ref, q_ref, dist_ref, thr_ref):
    keys = keys_ref[0]
    q = q_ref[0]
    n = keys.shape[0]
    inner_t = lax.dot_general(
        keys, q, (((1,), (1,)), ((), ())),
        preferred_element_type=jnp.float32)
    qn = jnp.sum(q * q, axis=1)
    kn = jnp.sum(keys * keys, axis=1)
    d_t = kn[:, None] - 2.0 * inner_t + qn[None, :]
    dist_ref[...] = d_t
    gm = jnp.stack(
        [jnp.min(d_t[128 * g:128 * (g + 1), :], axis=0)
         for g in range(n // 128)], axis=0)
    big = jnp.float32(jnp.inf)
    t = jnp.full((1, _R), -big, jnp.float32)
    for _ in range(_K):
        t = jnp.min(jnp.where(gm > t, gm, big), axis=0, keepdims=True)
    thr_ref[0, 0] = t[0]


def _sc_body(dist_hbm, thr_hbm, out_hbm, buf0, buf1, idxc, valc, outv, thrv,
             sem0, sem1):
    n = dist_hbm.shape[0]
    hcols = n // 2
    rows_w = thrv.shape[0]
    ngroups = rows_w // 16
    wid = lax.axis_index("s") * 2 + lax.axis_index("c")
    base_row = wid * rows_w
    iota16 = lax.broadcasted_iota(jnp.int32, (16,), 0)
    zeros16 = jnp.zeros((16,), jnp.int32)
    inf16 = jnp.full((16,), jnp.inf, jnp.float32)
    big = jnp.float32(jnp.inf)
    lane_cap = iota16 * _CAP

    pltpu.sync_copy(thr_hbm.at[pl.ds(base_row, rows_w)], thrv)

    def _src(g, h):
        return dist_hbm.at[pl.ds(h * hcols, hcols),
                           pl.ds(base_row + 16 * g, 16)]

    pltpu.async_copy(_src(0, 0), buf0, sem0)

    def _bmin(x):
        for sh in (1, 2, 4, 8):
            x = jnp.minimum(x, jnp.take(x, iota16 ^ sh))
        return x

    def _scan_half(buf, h, tvec, curs):
        def col4(i, curs):
            for k in range(4):
                cloc = 4 * i + k
                v = buf[cloc]
                mask = (v <= tvec) & (curs < _CAP)
                mi = jnp.where(mask, 1, 0).astype(jnp.int32)
                destpos = lane_cap + curs
                cg = zeros16 + (h * hcols + cloc)
                plsc.store_scatter(idxc, [destpos], cg, mask=mask)
                plsc.store_scatter(valc, [destpos], v, mask=mask)
                curs = curs + mi
            return curs

        return lax.fori_loop(0, hcols // 4, col4, curs)

    def _select_row(g, r, curs):
        cnt = jnp.take(curs, zeros16 + r)[0]
        rb = r * _CAP
        nb = lax.div(cnt + 15, 16)
        toff = rb + 16 * (nb - 1)
        tv = valc[pl.ds(toff, 16)]
        valc[pl.ds(toff, 16)] = jnp.where(16 * (nb - 1) + iota16 >= cnt,
                                          big, tv)

        def round_body(rr, resvec):
            def sel_scan(cc, st):
                acc, cid = st
                ch = valc[pl.ds(rb + 16 * cc, 16)]
                lt = ch < acc
                return jnp.where(lt, ch, acc), jnp.where(lt, cc, cid)

            acc, cid = lax.fori_loop(0, nb, sel_scan, (inf16, zeros16))
            m = _bmin(acc)
            posl = cid * 16 + iota16
            cand = jnp.where(acc == m, posl, jnp.int32(2 ** 30))
            pos = _bmin(cand) + rb
            idxsel = plsc.load_gather(idxc, [pos])
            plsc.store_scatter(valc, [pos], inf16, mask=iota16 == 0)
            return jnp.where(iota16 == rr, idxsel, resvec)

        resvec = lax.fori_loop(0, _K, round_body, zeros16)
        outv[pl.ds((16 * g + r) * _K, 16)] = resvec

    def group_body(g, _):
        tvec = thrv[pl.ds(16 * g, 16)]
        pltpu.make_async_copy(_src(g, 0), buf0, sem0).wait()
        pltpu.async_copy(_src(g, 1), buf1, sem1)
        curs = _scan_half(buf0, 0, tvec, zeros16)
        pltpu.make_async_copy(_src(g, 1), buf1, sem1).wait()

        @pl.when(g + 1 < ngroups)
        def _():
            pltpu.async_copy(_src(g + 1, 0), buf0, sem0)

        curs = _scan_half(buf1, 1, tvec, curs)
        for r in range(16):
            _select_row(g, r, curs)
        return 0

    lax.fori_loop(0, ngroups, group_body, 0)
    pltpu.sync_copy(outv, out_hbm.at[pl.ds(base_row * _K, rows_w * _K)])


def kernel(inputs):
    b, n, d = inputs.shape
    dist, thr = pl.pallas_call(
        _tc_body,
        grid=(b, n // _R),
        in_specs=[
            pl.BlockSpec((1, n, d), lambda bi, ri: (bi, 0, 0)),
            pl.BlockSpec((1, _R, d), lambda bi, ri: (bi, ri, 0)),
        ],
        out_specs=[
            pl.BlockSpec((n, _R), lambda bi, ri: (0, bi * (n // _R) + ri)),
            pl.BlockSpec((1, 1, _R), lambda bi, ri: (bi * (n // _R) + ri, 0, 0)),
        ],
        out_shape=[
            jax.ShapeDtypeStruct((n, b * n), jnp.float32),
            jax.ShapeDtypeStruct((b * n // _R, 1, _R), jnp.float32),
        ],
        compiler_params=pltpu.CompilerParams(
            dimension_semantics=("parallel", "arbitrary")),
    )(inputs, inputs)

    rows_w = b * n // _NW
    mesh = plsc.VectorSubcoreMesh(core_axis_name="c", subcore_axis_name="s",
                                  num_cores=2, num_subcores=16)
    sc_fn = functools.partial(
        pl.kernel,
        out_type=jax.ShapeDtypeStruct((b * n * _K,), jnp.int32),
        mesh=mesh,
        scratch_types=[
            pltpu.VMEM((n // 2, 16), jnp.float32),
            pltpu.VMEM((n // 2, 16), jnp.float32),
            pltpu.VMEM((16 * _CAP,), jnp.int32),
            pltpu.VMEM((16 * _CAP,), jnp.float32),
            pltpu.VMEM((rows_w * _K,), jnp.int32),
            pltpu.VMEM((rows_w,), jnp.float32),
            pltpu.SemaphoreType.DMA,
            pltpu.SemaphoreType.DMA,
        ],
        compiler_params=pltpu.CompilerParams(use_tc_tiling_on_sc=False,
                                             needs_layout_passes=False),
    )(_sc_body)
    out = sc_fn(dist, thr.reshape(b * n))
    return out.reshape(b, n, _K)


# speedup vs baseline: 1.1857x; 1.1857x over previous
import functools

import jax
import jax.numpy as jnp
from jax import lax
from jax.experimental import pallas as pl
from jax.experimental.pallas import tpu as pltpu
from jax.experimental.pallas import tpu_sc as plsc

_K = 16
_R = 256
_CAP = 128
_NW = 32
_SL = 256


def _tc_body(keys_ref, q_ref, dist_ref, thr_ref):
    keys = keys_ref[0]
    q = q_ref[0]
    n = keys.shape[0]
    inner_t = lax.dot_general(
        keys, q, (((1,), (1,)), ((), ())),
        preferred_element_type=jnp.float32)
    qn = jnp.sum(q * q, axis=1)
    kn = jnp.sum(keys * keys, axis=1)
    d_t = kn[:, None] - 2.0 * inner_t + qn[None, :]
    dist_ref[...] = d_t
    gm = jnp.stack(
        [jnp.min(d_t[128 * g:128 * (g + 1), :], axis=0)
         for g in range(n // 128)], axis=0)
    big = jnp.float32(jnp.inf)
    t = jnp.full((1, _R), -big, jnp.float32)
    for _ in range(_K):
        t = jnp.min(jnp.where(gm > t, gm, big), axis=0, keepdims=True)
    thr_ref[0, 0] = t[0]


def _sc_body(dist_hbm, thr_hbm, out_hbm, buf0, buf1, idxc, valc, cursb, outv,
             thrv, sem0, sem1):
    n = dist_hbm.shape[0]
    nsl = n // _SL
    rows_w = thrv.shape[0]
    ngroups = rows_w // 128
    wid = lax.axis_index("s") * 2 + lax.axis_index("c")
    base_row = wid * rows_w
    iota16 = lax.broadcasted_iota(jnp.int32, (16,), 0)
    zeros16 = jnp.zeros((16,), jnp.int32)
    inf16 = jnp.full((16,), jnp.inf, jnp.float32)
    big = jnp.float32(jnp.inf)
    subb = [(iota16 + 16 * s) * _CAP for s in range(8)]

    pltpu.sync_copy(thr_hbm.at[pl.ds(base_row, rows_w)], thrv)

    def _src(g, t):
        return dist_hbm.at[pl.ds(t * _SL, _SL),
                           pl.ds(base_row + 128 * g, 128)]

    pltpu.async_copy(_src(0, 0), buf0, sem0)

    def _bmin(x):
        for sh in (1, 2, 4, 8):
            x = jnp.minimum(x, jnp.take(x, iota16 ^ sh))
        return x

    def _scan_slice(buf, t, tvecs, curs):
        def col_body(i, curs):
            cg = zeros16 + (t * _SL + i)
            new = []
            for s in range(8):
                v = buf[i, pl.ds(16 * s, 16)]
                mask = v <= tvecs[s]
                destpos = subb[s] + jnp.minimum(curs[s], _CAP - 1)
                plsc.store_scatter(idxc, [destpos], cg, mask=mask)
                plsc.store_scatter(valc, [destpos], v, mask=mask)
                new.append(curs[s] + jnp.where(mask, 1, 0).astype(jnp.int32))
            return tuple(new)

        return lax.fori_loop(0, _SL, col_body, curs)

    def group_body(g, _):
        tvecs = [thrv[pl.ds(128 * g + 16 * s, 16)] for s in range(8)]

        def pair_body(p, curs):
            t0 = 2 * p
            pltpu.make_async_copy(_src(g, t0), buf0, sem0).wait()
            pltpu.async_copy(_src(g, t0 + 1), buf1, sem1)
            curs = _scan_slice(buf0, t0, tvecs, curs)
            pltpu.make_async_copy(_src(g, t0 + 1), buf1, sem1).wait()

            @pl.when(p + 1 < nsl // 2)
            def _():
                pltpu.async_copy(_src(g, t0 + 2), buf0, sem0)

            @pl.when((p + 1 == nsl // 2) & (g + 1 < ngroups))
            def _():
                pltpu.async_copy(_src(g + 1, 0), buf0, sem0)

            return _scan_slice(buf1, t0 + 1, tvecs, curs)

        curs = lax.fori_loop(0, nsl // 2, pair_body,
                             tuple(zeros16 for _ in range(8)))
        for s in range(8):
            cursb[pl.ds(16 * s, 16)] = jnp.minimum(curs[s], _CAP)

        def row_body(r, _):
            lane = lax.rem(r, 16)
            cch = cursb[pl.ds(r - lane, 16)]
            cnt = jnp.take(cch, zeros16 + lane)[0]
            rb = r * _CAP
            nb = lax.div(cnt + 15, 16)
            toff = rb + 16 * (nb - 1)
            tv = valc[pl.ds(toff, 16)]
            valc[pl.ds(toff, 16)] = jnp.where(16 * (nb - 1) + iota16 >= cnt,
                                              big, tv)

            def round_body(rr, resvec):
                def sel_scan(cc, st):
                    acc, cid = st
                    ch = valc[pl.ds(rb + 16 * cc, 16)]
                    lt = ch < acc
                    return jnp.where(lt, ch, acc), jnp.where(lt, cc, cid)

                acc, cid = lax.fori_loop(0, nb, sel_scan, (inf16, zeros16))
                m = _bmin(acc)
                posl = cid * 16 + iota16
                cand = jnp.where(acc == m, posl, jnp.int32(2 ** 30))
                pos = _bmin(cand) + rb
                idxsel = plsc.load_gather(idxc, [pos])
                plsc.store_scatter(valc, [pos], inf16, mask=iota16 == 0)
                return jnp.where(iota16 == rr, idxsel, resvec)

            resvec = lax.fori_loop(0, _K, round_body, zeros16)
            outv[pl.ds((128 * g + r) * _K, 16)] = resvec
            return 0

        lax.fori_loop(0, 128, row_body, 0)
        return 0

    lax.fori_loop(0, ngroups, group_body, 0)
    pltpu.sync_copy(outv, out_hbm.at[pl.ds(base_row * _K, rows_w * _K)])


def kernel(inputs):
    b, n, d = inputs.shape
    dist, thr = pl.pallas_call(
        _tc_body,
        grid=(b, n // _R),
        in_specs=[
            pl.BlockSpec((1, n, d), lambda bi, ri: (bi, 0, 0)),
            pl.BlockSpec((1, _R, d), lambda bi, ri: (bi, ri, 0)),
        ],
        out_specs=[
            pl.BlockSpec((n, _R), lambda bi, ri: (0, bi * (n // _R) + ri)),
            pl.BlockSpec((1, 1, _R), lambda bi, ri: (bi * (n // _R) + ri, 0, 0)),
        ],
        out_shape=[
            jax.ShapeDtypeStruct((n, b * n), jnp.float32),
            jax.ShapeDtypeStruct((b * n // _R, 1, _R), jnp.float32),
        ],
        compiler_params=pltpu.CompilerParams(
            dimension_semantics=("parallel", "arbitrary")),
    )(inputs, inputs)

    rows_w = b * n // _NW
    mesh = plsc.VectorSubcoreMesh(core_axis_name="c", subcore_axis_name="s",
                                  num_cores=2, num_subcores=16)
    sc_fn = functools.partial(
        pl.kernel,
        out_type=jax.ShapeDtypeStruct((b * n * _K,), jnp.int32),
        mesh=mesh,
        scratch_types=[
            pltpu.VMEM((_SL, 128), jnp.float32),
            pltpu.VMEM((_SL, 128), jnp.float32),
            pltpu.VMEM((128 * _CAP,), jnp.int32),
            pltpu.VMEM((128 * _CAP,), jnp.float32),
            pltpu.VMEM((128,), jnp.int32),
            pltpu.VMEM((rows_w * _K,), jnp.int32),
            pltpu.VMEM((rows_w,), jnp.float32),
            pltpu.SemaphoreType.DMA,
            pltpu.SemaphoreType.DMA,
        ],
        compiler_params=pltpu.CompilerParams(use_tc_tiling_on_sc=True,
                                             needs_layout_passes=False),
    )(_sc_body)
    out = sc_fn(dist, thr.reshape(b * n))
    return out.reshape(b, n, _K)


# speedup vs baseline: 2.2809x; 1.9237x over previous
import functools

import jax
import jax.numpy as jnp
from jax import lax
from jax.experimental import pallas as pl
from jax.experimental.pallas import tpu as pltpu
from jax.experimental.pallas import tpu_sc as plsc

_K = 16
_R = 256
_CAP = 128
_NW = 32
_SL = 256


def _tc_body(keys_ref, q_ref, dist_ref, thr_ref):
    keys = keys_ref[0]
    q = q_ref[0]
    n = keys.shape[0]
    inner_t = lax.dot_general(
        keys, q, (((1,), (1,)), ((), ())),
        preferred_element_type=jnp.float32)
    qn = jnp.sum(q * q, axis=1)
    kn = jnp.sum(keys * keys, axis=1)
    d_t = kn[:, None] - 2.0 * inner_t + qn[None, :]
    dist_ref[...] = d_t
    gm = jnp.stack(
        [jnp.min(d_t[128 * g:128 * (g + 1), :], axis=0)
         for g in range(n // 128)], axis=0)
    big = jnp.float32(jnp.inf)
    t = jnp.full((1, _R), -big, jnp.float32)
    for _ in range(_K):
        t = jnp.min(jnp.where(gm > t, gm, big), axis=0, keepdims=True)
    thr_ref[0, 0] = t[0]


def _sc_body(dist_hbm, thr_hbm, out_hbm, buf0, buf1, idxc, valc, cursb, outv,
             thrv, sem0, sem1):
    n = dist_hbm.shape[0]
    nsl = n // _SL
    rows_w = thrv.shape[0]
    ngroups = rows_w // 128
    wid = lax.axis_index("s") * 2 + lax.axis_index("c")
    base_row = wid * rows_w
    iota16 = lax.broadcasted_iota(jnp.int32, (16,), 0)
    zeros16 = jnp.zeros((16,), jnp.int32)
    inf16 = jnp.full((16,), jnp.inf, jnp.float32)
    big = jnp.float32(jnp.inf)
    subb = [(iota16 + 16 * s) * _CAP for s in range(8)]

    pltpu.sync_copy(thr_hbm.at[pl.ds(base_row, rows_w)], thrv)

    def _src(g, t):
        return dist_hbm.at[pl.ds(t * _SL, _SL),
                           pl.ds(base_row + 128 * g, 128)]

    pltpu.async_copy(_src(0, 0), buf0, sem0)

    def _bmin(x):
        for sh in (1, 2, 4, 8):
            x = jnp.minimum(x, jnp.take(x, iota16 ^ sh))
        return x

    def _scan_slice(buf, t, tvecs, curs):
        def col_body(i, curs):
            cg = zeros16 + (t * _SL + i)
            vs = [buf[i, pl.ds(16 * s, 16)] for s in range(8)]
            masks = [vs[s] <= tvecs[s] for s in range(8)]
            dests = [subb[s] + jnp.minimum(curs[s], _CAP - 1)
                     for s in range(8)]
            for s in range(8):
                plsc.store_scatter(idxc, [dests[s]], cg, mask=masks[s])
                plsc.store_scatter(valc, [dests[s]], vs[s], mask=masks[s])
            return tuple(
                curs[s] + jnp.where(masks[s], 1, 0).astype(jnp.int32)
                for s in range(8))

        return lax.fori_loop(0, _SL, col_body, curs)

    def group_body(g, _):
        tvecs = [thrv[pl.ds(128 * g + 16 * s, 16)] for s in range(8)]

        def pair_body(p, curs):
            t0 = 2 * p
            pltpu.make_async_copy(_src(g, t0), buf0, sem0).wait()
            pltpu.async_copy(_src(g, t0 + 1), buf1, sem1)
            curs = _scan_slice(buf0, t0, tvecs, curs)
            pltpu.make_async_copy(_src(g, t0 + 1), buf1, sem1).wait()

            @pl.when(p + 1 < nsl // 2)
            def _():
                pltpu.async_copy(_src(g, t0 + 2), buf0, sem0)

            @pl.when((p + 1 == nsl // 2) & (g + 1 < ngroups))
            def _():
                pltpu.async_copy(_src(g + 1, 0), buf0, sem0)

            return _scan_slice(buf1, t0 + 1, tvecs, curs)

        curs = lax.fori_loop(0, nsl // 2, pair_body,
                             tuple(zeros16 for _ in range(8)))
        for s in range(8):
            cursb[pl.ds(16 * s, 16)] = jnp.minimum(curs[s], _CAP)

        def row_body(r, _):
            lane = lax.rem(r, 16)
            cch = cursb[pl.ds(r - lane, 16)]
            cnt = jnp.take(cch, zeros16 + lane)[0]
            rb = r * _CAP
            nb = lax.div(cnt + 15, 16)
            toff = rb + 16 * (nb - 1)
            tv = valc[pl.ds(toff, 16)]
            valc[pl.ds(toff, 16)] = jnp.where(16 * (nb - 1) + iota16 >= cnt,
                                              big, tv)

            def round_body(rr, resvec):
                def sel_scan(cc, st):
                    acc, cid = st
                    ch = valc[pl.ds(rb + 16 * cc, 16)]
                    lt = ch < acc
                    return jnp.where(lt, ch, acc), jnp.where(lt, cc, cid)

                acc, cid = lax.fori_loop(0, nb, sel_scan, (inf16, zeros16))
                m = _bmin(acc)
                posl = cid * 16 + iota16
                cand = jnp.where(acc == m, posl, jnp.int32(2 ** 30))
                pos = _bmin(cand) + rb
                idxsel = plsc.load_gather(idxc, [pos])
                plsc.store_scatter(valc, [pos], inf16, mask=iota16 == 0)
                return jnp.where(iota16 == rr, idxsel, resvec)

            resvec = lax.fori_loop(0, _K, round_body, zeros16)
            outv[pl.ds((128 * g + r) * _K, 16)] = resvec
            return 0

        lax.fori_loop(0, 128, row_body, 0)
        return 0

    lax.fori_loop(0, ngroups, group_body, 0)
    pltpu.sync_copy(outv, out_hbm.at[pl.ds(base_row * _K, rows_w * _K)])


def kernel(inputs):
    b, n, d = inputs.shape
    dist, thr = pl.pallas_call(
        _tc_body,
        grid=(b, n // _R),
        in_specs=[
            pl.BlockSpec((1, n, d), lambda bi, ri: (bi, 0, 0)),
            pl.BlockSpec((1, _R, d), lambda bi, ri: (bi, ri, 0)),
        ],
        out_specs=[
            pl.BlockSpec((n, _R), lambda bi, ri: (0, bi * (n // _R) + ri)),
            pl.BlockSpec((1, 1, _R), lambda bi, ri: (bi * (n // _R) + ri, 0, 0)),
        ],
        out_shape=[
            jax.ShapeDtypeStruct((n, b * n), jnp.float32),
            jax.ShapeDtypeStruct((b * n // _R, 1, _R), jnp.float32),
        ],
        compiler_params=pltpu.CompilerParams(
            dimension_semantics=("parallel", "arbitrary")),
    )(inputs, inputs)

    rows_w = b * n // _NW
    mesh = plsc.VectorSubcoreMesh(core_axis_name="c", subcore_axis_name="s",
                                  num_cores=2, num_subcores=16)
    sc_fn = functools.partial(
        pl.kernel,
        out_type=jax.ShapeDtypeStruct((b * n * _K,), jnp.int32),
        mesh=mesh,
        scratch_types=[
            pltpu.VMEM((_SL, 128), jnp.float32),
            pltpu.VMEM((_SL, 128), jnp.float32),
            pltpu.VMEM((128 * _CAP,), jnp.int32),
            pltpu.VMEM((128 * _CAP,), jnp.float32),
            pltpu.VMEM((128,), jnp.int32),
            pltpu.VMEM((rows_w * _K,), jnp.int32),
            pltpu.VMEM((rows_w,), jnp.float32),
            pltpu.SemaphoreType.DMA,
            pltpu.SemaphoreType.DMA,
        ],
        compiler_params=pltpu.CompilerParams(use_tc_tiling_on_sc=True,
                                             needs_layout_passes=False),
    )(_sc_body)
    out = sc_fn(dist, thr.reshape(b * n))
    return out.reshape(b, n, _K)


# speedup vs baseline: 3.2810x; 1.4385x over previous
import functools

import jax
import jax.numpy as jnp
from jax import lax
from jax.experimental import pallas as pl
from jax.experimental.pallas import tpu as pltpu
from jax.experimental.pallas import tpu_sc as plsc

_K = 16
_R = 256
_CAP = 128
_NW = 32
_SL = 256


def _tc_body(keys_ref, q_ref, dist_ref, thr_ref):
    keys = keys_ref[0]
    q = q_ref[0]
    n = keys.shape[0]
    inner_t = lax.dot_general(
        keys, q, (((1,), (1,)), ((), ())),
        preferred_element_type=jnp.float32)
    qn = jnp.sum(q * q, axis=1)
    kn = jnp.sum(keys * keys, axis=1)
    d_t = kn[:, None] - 2.0 * inner_t + qn[None, :]
    dist_ref[...] = d_t
    gm = jnp.stack(
        [jnp.min(d_t[128 * g:128 * (g + 1), :], axis=0)
         for g in range(n // 128)], axis=0)
    big = jnp.float32(jnp.inf)
    t = jnp.full((1, _R), -big, jnp.float32)
    for _ in range(_K):
        t = jnp.min(jnp.where(gm > t, gm, big), axis=0, keepdims=True)
    thr_ref[0, 0] = t[0]


def _sc_body(dist_hbm, thr_hbm, out_hbm, buf0, buf1, idxc, valc, cursb, outv,
             thrv, sem0, sem1):
    n = dist_hbm.shape[0]
    nsl = n // _SL
    rows_w = thrv.shape[0]
    ngroups = rows_w // 128
    wid = lax.axis_index("s") * 2 + lax.axis_index("c")
    base_row = wid * rows_w
    iota16 = lax.broadcasted_iota(jnp.int32, (16,), 0)
    zeros16 = jnp.zeros((16,), jnp.int32)
    inf16 = jnp.full((16,), jnp.inf, jnp.float32)
    big = jnp.float32(jnp.inf)
    subb = [(iota16 + 16 * s) * _CAP for s in range(8)]

    pltpu.sync_copy(thr_hbm.at[pl.ds(base_row, rows_w)], thrv)

    def _src(g, t):
        return dist_hbm.at[pl.ds(t * _SL, _SL),
                           pl.ds(base_row + 128 * g, 128)]

    pltpu.async_copy(_src(0, 0), buf0, sem0)

    def _bmin(x):
        for sh in (1, 2, 4, 8):
            x = jnp.minimum(x, jnp.take(x, iota16 ^ sh))
        return x

    def _scan_slice(buf, t, tvecs, curs):
        def col_body(i, curs):
            cg = zeros16 + (t * _SL + i)
            vs = [buf[i, pl.ds(16 * s, 16)] for s in range(8)]
            masks = [vs[s] <= tvecs[s] for s in range(8)]
            dests = [subb[s] + jnp.minimum(curs[s], _CAP - 1)
                     for s in range(8)]
            for s in range(8):
                plsc.store_scatter(idxc, [dests[s]], cg, mask=masks[s])
                plsc.store_scatter(valc, [dests[s]], vs[s], mask=masks[s])
            return tuple(
                curs[s] + jnp.where(masks[s], 1, 0).astype(jnp.int32)
                for s in range(8))

        return lax.fori_loop(0, _SL, col_body, curs)

    def group_body(g, _):
        tvecs = [thrv[pl.ds(128 * g + 16 * s, 16)] for s in range(8)]

        def pair_body(p, curs):
            t0 = 2 * p
            pltpu.make_async_copy(_src(g, t0), buf0, sem0).wait()
            pltpu.async_copy(_src(g, t0 + 1), buf1, sem1)
            curs = _scan_slice(buf0, t0, tvecs, curs)
            pltpu.make_async_copy(_src(g, t0 + 1), buf1, sem1).wait()

            @pl.when(p + 1 < nsl // 2)
            def _():
                pltpu.async_copy(_src(g, t0 + 2), buf0, sem0)

            @pl.when((p + 1 == nsl // 2) & (g + 1 < ngroups))
            def _():
                pltpu.async_copy(_src(g + 1, 0), buf0, sem0)

            return _scan_slice(buf1, t0 + 1, tvecs, curs)

        curs = lax.fori_loop(0, nsl // 2, pair_body,
                             tuple(zeros16 for _ in range(8)))
        for s in range(8):
            cursb[pl.ds(16 * s, 16)] = jnp.minimum(curs[s], _CAP)

        def pack_body(pk, _):
            rows = [4 * pk + k for k in range(4)]
            rbs, nbs, accs, cids = [], [], [], []
            for r in rows:
                lane = lax.rem(r, 16)
                cch = cursb[pl.ds(r - lane, 16)]
                cnt = jnp.take(cch, zeros16 + lane)[0]
                rb = r * _CAP
                nb = lax.div(cnt + 15, 16)
                toff = rb + 16 * (nb - 1)
                tv = valc[pl.ds(toff, 16)]
                valc[pl.ds(toff, 16)] = jnp.where(
                    16 * (nb - 1) + iota16 >= cnt, big, tv)

                def sel_scan(cc, st, rb=rb):
                    acc, cid = st
                    ch = valc[pl.ds(rb + 16 * cc, 16)]
                    lt = ch < acc
                    return jnp.where(lt, ch, acc), jnp.where(lt, cc, cid)

                acc, cid = lax.fori_loop(0, nb, sel_scan, (inf16, zeros16))
                rbs.append(rb)
                nbs.append(nb)
                accs.append(acc)
                cids.append(cid)

            def round_body(rr, st):
                res = list(st[0:4])
                acc = list(st[4:8])
                cid = list(st[8:12])
                for k in range(4):
                    m = jnp.min(acc[k])
                    cand = jnp.where(acc[k] == m, cid[k] * 16 + iota16,
                                     jnp.int32(2 ** 30))
                    p = jnp.min(cand)
                    lane_s = lax.rem(p, 16)
                    pos = rbs[k] + p - lane_s + lane_s
                    possplat = zeros16 + pos
                    idxsel = plsc.load_gather(idxc, [possplat])
                    res[k] = jnp.where(iota16 == rr, idxsel, res[k])
                    plsc.store_scatter(valc, [possplat], inf16,
                                       mask=iota16 == 0)
                    lv = plsc.load_gather(
                        valc, [rbs[k] + iota16 * 16 + lane_s])
                    lv = jnp.where(iota16 < nbs[k], lv, big)
                    nm = jnp.min(lv)
                    ncid = jnp.min(jnp.where(lv == nm, iota16,
                                             jnp.int32(2 ** 30)))
                    lmask = iota16 == lane_s
                    acc[k] = jnp.where(lmask, nm, acc[k])
                    cid[k] = jnp.where(lmask, ncid, cid[k])
                return tuple(res + acc + cid)

            st = tuple([zeros16] * 4 + accs + cids)
            st = lax.fori_loop(0, _K, round_body, st)
            for k in range(4):
                outv[pl.ds((128 * g + rows[k]) * _K, 16)] = st[k]
            return 0

        lax.fori_loop(0, 32, pack_body, 0)
        return 0

    lax.fori_loop(0, ngroups, group_body, 0)
    pltpu.sync_copy(outv, out_hbm.at[pl.ds(base_row * _K, rows_w * _K)])


def kernel(inputs):
    b, n, d = inputs.shape
    dist, thr = pl.pallas_call(
        _tc_body,
        grid=(b, n // _R),
        in_specs=[
            pl.BlockSpec((1, n, d), lambda bi, ri: (bi, 0, 0)),
            pl.BlockSpec((1, _R, d), lambda bi, ri: (bi, ri, 0)),
        ],
        out_specs=[
            pl.BlockSpec((n, _R), lambda bi, ri: (0, bi * (n // _R) + ri)),
            pl.BlockSpec((1, 1, _R), lambda bi, ri: (bi * (n // _R) + ri, 0, 0)),
        ],
        out_shape=[
            jax.ShapeDtypeStruct((n, b * n), jnp.float32),
            jax.ShapeDtypeStruct((b * n // _R, 1, _R), jnp.float32),
        ],
        compiler_params=pltpu.CompilerParams(
            dimension_semantics=("parallel", "arbitrary")),
    )(inputs, inputs)

    rows_w = b * n // _NW
    mesh = plsc.VectorSubcoreMesh(core_axis_name="c", subcore_axis_name="s",
                                  num_cores=2, num_subcores=16)
    sc_fn = functools.partial(
        pl.kernel,
        out_type=jax.ShapeDtypeStruct((b * n * _K,), jnp.int32),
        mesh=mesh,
        scratch_types=[
            pltpu.VMEM((_SL, 128), jnp.float32),
            pltpu.VMEM((_SL, 128), jnp.float32),
            pltpu.VMEM((128 * _CAP,), jnp.int32),
            pltpu.VMEM((128 * _CAP,), jnp.float32),
            pltpu.VMEM((128,), jnp.int32),
            pltpu.VMEM((rows_w * _K,), jnp.int32),
            pltpu.VMEM((rows_w,), jnp.float32),
            pltpu.SemaphoreType.DMA,
            pltpu.SemaphoreType.DMA,
        ],
        compiler_params=pltpu.CompilerParams(use_tc_tiling_on_sc=True,
                                             needs_layout_passes=False),
    )(_sc_body)
    out = sc_fn(dist, thr.reshape(b * n))
    return out.reshape(b, n, _K)


# speedup vs baseline: 3.7419x; 1.1405x over previous
import functools

import jax
import jax.numpy as jnp
from jax import lax
from jax.experimental import pallas as pl
from jax.experimental.pallas import tpu as pltpu
from jax.experimental.pallas import tpu_sc as plsc

_K = 16
_R = 256
_CAP = 128
_NW = 32
_SL = 256


def _tc_body(keys_ref, q_ref, dist_ref, thr_ref):
    keys = keys_ref[0]
    q = q_ref[0]
    n = keys.shape[0]
    inner_t = lax.dot_general(
        keys, q, (((1,), (1,)), ((), ())),
        preferred_element_type=jnp.float32)
    qn = jnp.sum(q * q, axis=1)
    kn = jnp.sum(keys * keys, axis=1)
    d_t = kn[:, None] - 2.0 * inner_t + qn[None, :]
    dist_ref[...] = d_t
    gm = jnp.stack(
        [jnp.min(d_t[128 * g:128 * (g + 1), :], axis=0)
         for g in range(n // 128)], axis=0)
    big = jnp.float32(jnp.inf)
    t = jnp.full((1, _R), -big, jnp.float32)
    for _ in range(_K):
        t = jnp.min(jnp.where(gm > t, gm, big), axis=0, keepdims=True)
    thr_ref[0, 0] = t[0]


def _sc_body(dist_hbm, thr_hbm, out_hbm, buf0, buf1, idxc, valc, cursb, outv,
             thrv, sem0, sem1):
    n = dist_hbm.shape[0]
    nsl = n // _SL
    rows_w = thrv.shape[0]
    ngroups = rows_w // 128
    wid = lax.axis_index("s") * 2 + lax.axis_index("c")
    base_row = wid * rows_w
    iota16 = lax.broadcasted_iota(jnp.int32, (16,), 0)
    zeros16 = jnp.zeros((16,), jnp.int32)
    inf16 = jnp.full((16,), jnp.inf, jnp.float32)
    big = jnp.float32(jnp.inf)
    subb = [(iota16 + 16 * s) * _CAP for s in range(8)]

    pltpu.sync_copy(thr_hbm.at[pl.ds(base_row, rows_w)], thrv)

    def _src(g, t):
        return dist_hbm.at[pl.ds(t * _SL, _SL),
                           pl.ds(base_row + 128 * g, 128)]

    pltpu.async_copy(_src(0, 0), buf0, sem0)

    def _bmin(x):
        for sh in (1, 2, 4, 8):
            x = jnp.minimum(x, jnp.take(x, iota16 ^ sh))
        return x

    def _scan_slice(buf, t, tvecs, curs):
        def col_body(i, curs):
            cg = zeros16 + (t * _SL + i)
            vs = [buf[i, pl.ds(16 * s, 16)] for s in range(8)]
            masks = [vs[s] <= tvecs[s] for s in range(8)]
            dests = [subb[s] + jnp.minimum(curs[s], _CAP - 1)
                     for s in range(8)]
            for s in range(8):
                plsc.store_scatter(idxc, [dests[s]], cg, mask=masks[s])
                plsc.store_scatter(valc, [dests[s]], vs[s], mask=masks[s])
            return tuple(
                curs[s] + jnp.where(masks[s], 1, 0).astype(jnp.int32)
                for s in range(8))

        return lax.fori_loop(0, _SL, col_body, curs)

    def group_body(g, _):
        tvecs = [thrv[pl.ds(128 * g + 16 * s, 16)] for s in range(8)]

        def pair_body(p, curs):
            t0 = 2 * p
            pltpu.make_async_copy(_src(g, t0), buf0, sem0).wait()
            pltpu.async_copy(_src(g, t0 + 1), buf1, sem1)
            curs = _scan_slice(buf0, t0, tvecs, curs)
            pltpu.make_async_copy(_src(g, t0 + 1), buf1, sem1).wait()

            @pl.when(p + 1 < nsl // 2)
            def _():
                pltpu.async_copy(_src(g, t0 + 2), buf0, sem0)

            @pl.when((p + 1 == nsl // 2) & (g + 1 < ngroups))
            def _():
                pltpu.async_copy(_src(g + 1, 0), buf0, sem0)

            return _scan_slice(buf1, t0 + 1, tvecs, curs)

        curs = lax.fori_loop(0, nsl // 2, pair_body,
                             tuple(zeros16 for _ in range(8)))
        for s in range(8):
            cursb[pl.ds(16 * s, 16)] = jnp.minimum(curs[s], _CAP)

        def pack_body(pk, _):
            rows = [4 * pk + k for k in range(4)]
            rbs, nbs, accs, cids = [], [], [], []
            for r in rows:
                lane = lax.rem(r, 16)
                cch = cursb[pl.ds(r - lane, 16)]
                cnt = jnp.take(cch, zeros16 + lane)[0]
                rb = r * _CAP
                nb = lax.div(cnt + 15, 16)
                toff = rb + 16 * (nb - 1)
                tv = valc[pl.ds(toff, 16)]
                valc[pl.ds(toff, 16)] = jnp.where(
                    16 * (nb - 1) + iota16 >= cnt, big, tv)

                def sel_scan(cc, st, rb=rb):
                    acc, cid = st
                    ch = valc[pl.ds(rb + 16 * cc, 16)]
                    lt = ch < acc
                    return jnp.where(lt, ch, acc), jnp.where(lt, cc, cid)

                acc, cid = lax.fori_loop(0, nb, sel_scan, (inf16, zeros16))
                rbs.append(rb)
                nbs.append(nb)
                accs.append(acc)
                cids.append(cid)

            def round_body(rr, st):
                res = list(st[0:4])
                acc = list(st[4:8])
                cid = list(st[8:12])
                for k in range(4):
                    m = jnp.min(acc[k])
                    cand = jnp.where(acc[k] == m, cid[k] * 16 + iota16,
                                     jnp.int32(2 ** 30))
                    p = jnp.min(cand)
                    lane_s = lax.rem(p, 16)
                    pos = rbs[k] + p - lane_s + lane_s
                    possplat = zeros16 + pos
                    idxsel = plsc.load_gather(idxc, [possplat])
                    res[k] = jnp.where(iota16 == rr, idxsel, res[k])
                    plsc.store_scatter(valc, [possplat], inf16,
                                       mask=iota16 == 0)
                    lv = plsc.load_gather(
                        valc, [rbs[k] + iota16 * 16 + lane_s])
                    lv = jnp.where(iota16 < nbs[k], lv, big)
                    nm = jnp.min(lv)
                    ncid = jnp.min(jnp.where(lv == nm, iota16,
                                             jnp.int32(2 ** 30)))
                    lmask = iota16 == lane_s
                    acc[k] = jnp.where(lmask, nm, acc[k])
                    cid[k] = jnp.where(lmask, ncid, cid[k])
                return tuple(res + acc + cid)

            st = tuple([zeros16] * 4 + accs + cids)
            st = lax.fori_loop(0, _K, round_body, st)
            for k in range(4):
                outv[pl.ds((128 * g + rows[k]) * _K, 16)] = st[k]
            return 0

        lax.fori_loop(0, 32, pack_body, 0)
        return 0

    lax.fori_loop(0, ngroups, group_body, 0)
    pltpu.sync_copy(outv, out_hbm.at[pl.ds(base_row * _K, rows_w * _K)])


def kernel(inputs):
    b, n, d = inputs.shape
    tc_fn = pl.pallas_call(
        _tc_body,
        grid=(1, n // _R),
        in_specs=[
            pl.BlockSpec((1, n, d), lambda bi, ri: (bi, 0, 0)),
            pl.BlockSpec((1, _R, d), lambda bi, ri: (bi, ri, 0)),
        ],
        out_specs=[
            pl.BlockSpec((n, _R), lambda bi, ri: (0, ri)),
            pl.BlockSpec((1, 1, _R), lambda bi, ri: (ri, 0, 0)),
        ],
        out_shape=[
            jax.ShapeDtypeStruct((n, n), jnp.float32),
            jax.ShapeDtypeStruct((n // _R, 1, _R), jnp.float32),
        ],
        compiler_params=pltpu.CompilerParams(
            dimension_semantics=("parallel", "arbitrary")),
    )

    rows_w = n // _NW
    mesh = plsc.VectorSubcoreMesh(core_axis_name="c", subcore_axis_name="s",
                                  num_cores=2, num_subcores=16)
    sc_fn = functools.partial(
        pl.kernel,
        out_type=jax.ShapeDtypeStruct((n * _K,), jnp.int32),
        mesh=mesh,
        scratch_types=[
            pltpu.VMEM((_SL, 128), jnp.float32),
            pltpu.VMEM((_SL, 128), jnp.float32),
            pltpu.VMEM((128 * _CAP,), jnp.int32),
            pltpu.VMEM((128 * _CAP,), jnp.float32),
            pltpu.VMEM((128,), jnp.int32),
            pltpu.VMEM((rows_w * _K,), jnp.int32),
            pltpu.VMEM((rows_w,), jnp.float32),
            pltpu.SemaphoreType.DMA,
            pltpu.SemaphoreType.DMA,
        ],
        compiler_params=pltpu.CompilerParams(use_tc_tiling_on_sc=True,
                                             needs_layout_passes=False),
    )(_sc_body)

    outs = []
    for bi in range(b):
        dist, thr = tc_fn(inputs[bi:bi + 1], inputs[bi:bi + 1])
        outs.append(sc_fn(dist, thr.reshape(n)))
    return jnp.stack(outs).reshape(b, n, _K)


# speedup vs baseline: 3.8032x; 1.0164x over previous
import functools

import jax
import jax.numpy as jnp
from jax import lax
from jax.experimental import pallas as pl
from jax.experimental.pallas import tpu as pltpu
from jax.experimental.pallas import tpu_sc as plsc

_K = 16
_R = 256
_CAP = 128
_NW = 32
_SL = 256


def _tc_body(keys_ref, q_ref, dist_ref, thr_ref):
    keys = keys_ref[0]
    q = q_ref[0]
    n = keys.shape[0]
    inner_t = lax.dot_general(
        keys, q, (((1,), (1,)), ((), ())),
        preferred_element_type=jnp.float32)
    qn = jnp.sum(q * q, axis=1)
    kn = jnp.sum(keys * keys, axis=1)
    d_t = kn[:, None] - 2.0 * inner_t + qn[None, :]
    dist_ref[...] = d_t
    gm = jnp.stack(
        [jnp.min(d_t[128 * g:128 * (g + 1), :], axis=0)
         for g in range(n // 128)], axis=0)
    big = jnp.float32(jnp.inf)
    t = jnp.full((1, _R), -big, jnp.float32)
    for _ in range(_K):
        t = jnp.min(jnp.where(gm > t, gm, big), axis=0, keepdims=True)
    thr_ref[0, 0] = t[0]


def _sc_body(dist_hbm, thr_hbm, out_hbm, buf0, buf1, idxc, valc, cursb, outv,
             thrv, sem0, sem1):
    n = dist_hbm.shape[0]
    nsl = n // _SL
    rows_w = thrv.shape[0]
    ngroups = rows_w // 128
    wid = lax.axis_index("s") * 2 + lax.axis_index("c")
    base_row = wid * rows_w
    iota16 = lax.broadcasted_iota(jnp.int32, (16,), 0)
    zeros16 = jnp.zeros((16,), jnp.int32)
    inf16 = jnp.full((16,), jnp.inf, jnp.float32)
    big = jnp.float32(jnp.inf)
    subb = [(iota16 + 16 * s) * _CAP for s in range(8)]

    pltpu.sync_copy(thr_hbm.at[pl.ds(base_row, rows_w)], thrv)

    def _src(g, t):
        return dist_hbm.at[pl.ds(t * _SL, _SL),
                           pl.ds(base_row + 128 * g, 128)]

    pltpu.async_copy(_src(0, 0), buf0, sem0)

    def _bmin(x):
        for sh in (1, 2, 4, 8):
            x = jnp.minimum(x, jnp.take(x, iota16 ^ sh))
        return x

    def _scan_slice(buf, t, tvecs, curs):
        def col_body(i, curs):
            for u in range(2):
                ic = 2 * i + u
                cg = zeros16 + (t * _SL + ic)
                vs = [buf[ic, pl.ds(16 * s, 16)] for s in range(8)]
                masks = [vs[s] <= tvecs[s] for s in range(8)]
                dests = [subb[s] + jnp.minimum(curs[s], _CAP - 1)
                         for s in range(8)]
                for s in range(8):
                    plsc.store_scatter(idxc, [dests[s]], cg, mask=masks[s])
                    plsc.store_scatter(valc, [dests[s]], vs[s], mask=masks[s])
                curs = tuple(
                    curs[s] + jnp.where(masks[s], 1, 0).astype(jnp.int32)
                    for s in range(8))
            return curs

        return lax.fori_loop(0, _SL // 2, col_body, curs)

    def group_body(g, _):
        tvecs = [thrv[pl.ds(128 * g + 16 * s, 16)] for s in range(8)]

        def pair_body(p, curs):
            t0 = 2 * p
            pltpu.make_async_copy(_src(g, t0), buf0, sem0).wait()
            pltpu.async_copy(_src(g, t0 + 1), buf1, sem1)
            curs = _scan_slice(buf0, t0, tvecs, curs)
            pltpu.make_async_copy(_src(g, t0 + 1), buf1, sem1).wait()

            @pl.when(p + 1 < nsl // 2)
            def _():
                pltpu.async_copy(_src(g, t0 + 2), buf0, sem0)

            @pl.when((p + 1 == nsl // 2) & (g + 1 < ngroups))
            def _():
                pltpu.async_copy(_src(g + 1, 0), buf0, sem0)

            return _scan_slice(buf1, t0 + 1, tvecs, curs)

        curs = lax.fori_loop(0, nsl // 2, pair_body,
                             tuple(zeros16 for _ in range(8)))
        for s in range(8):
            cursb[pl.ds(16 * s, 16)] = jnp.minimum(curs[s], _CAP)

        def pack_body(pk, _):
            rows = [4 * pk + k for k in range(4)]
            rbs, nbs, accs, cids = [], [], [], []
            for r in rows:
                lane = lax.rem(r, 16)
                cch = cursb[pl.ds(r - lane, 16)]
                cnt = jnp.take(cch, zeros16 + lane)[0]
                rb = r * _CAP
                nb = lax.div(cnt + 15, 16)
                toff = rb + 16 * (nb - 1)
                tv = valc[pl.ds(toff, 16)]
                valc[pl.ds(toff, 16)] = jnp.where(
                    16 * (nb - 1) + iota16 >= cnt, big, tv)

                def sel_scan(cc, st, rb=rb):
                    acc, cid = st
                    ch = valc[pl.ds(rb + 16 * cc, 16)]
                    lt = ch < acc
                    return jnp.where(lt, ch, acc), jnp.where(lt, cc, cid)

                acc, cid = lax.fori_loop(0, nb, sel_scan, (inf16, zeros16))
                rbs.append(rb)
                nbs.append(nb)
                accs.append(acc)
                cids.append(cid)

            def round_body(rr, st):
                res = list(st[0:4])
                acc = list(st[4:8])
                cid = list(st[8:12])
                for k in range(4):
                    m = jnp.min(acc[k])
                    cand = jnp.where(acc[k] == m, cid[k] * 16 + iota16,
                                     jnp.int32(2 ** 30))
                    p = jnp.min(cand)
                    lane_s = lax.rem(p, 16)
                    pos = rbs[k] + p - lane_s + lane_s
                    possplat = zeros16 + pos
                    idxsel = plsc.load_gather(idxc, [possplat])
                    res[k] = jnp.where(iota16 == rr, idxsel, res[k])
                    plsc.store_scatter(valc, [possplat], inf16,
                                       mask=iota16 == 0)
                    lv = plsc.load_gather(
                        valc, [rbs[k] + iota16 * 16 + lane_s])
                    lv = jnp.where(iota16 < nbs[k], lv, big)
                    nm = jnp.min(lv)
                    ncid = jnp.min(jnp.where(lv == nm, iota16,
                                             jnp.int32(2 ** 30)))
                    lmask = iota16 == lane_s
                    acc[k] = jnp.where(lmask, nm, acc[k])
                    cid[k] = jnp.where(lmask, ncid, cid[k])
                return tuple(res + acc + cid)

            st = tuple([zeros16] * 4 + accs + cids)
            st = lax.fori_loop(0, _K, round_body, st)
            for k in range(4):
                outv[pl.ds((128 * g + rows[k]) * _K, 16)] = st[k]
            return 0

        lax.fori_loop(0, 32, pack_body, 0)
        return 0

    lax.fori_loop(0, ngroups, group_body, 0)
    pltpu.sync_copy(outv, out_hbm.at[pl.ds(base_row * _K, rows_w * _K)])


def kernel(inputs):
    b, n, d = inputs.shape
    tc_fn = pl.pallas_call(
        _tc_body,
        grid=(1, n // _R),
        in_specs=[
            pl.BlockSpec((1, n, d), lambda bi, ri: (bi, 0, 0)),
            pl.BlockSpec((1, _R, d), lambda bi, ri: (bi, ri, 0)),
        ],
        out_specs=[
            pl.BlockSpec((n, _R), lambda bi, ri: (0, ri)),
            pl.BlockSpec((1, 1, _R), lambda bi, ri: (ri, 0, 0)),
        ],
        out_shape=[
            jax.ShapeDtypeStruct((n, n), jnp.float32),
            jax.ShapeDtypeStruct((n // _R, 1, _R), jnp.float32),
        ],
        compiler_params=pltpu.CompilerParams(
            dimension_semantics=("parallel", "arbitrary")),
    )

    rows_w = n // _NW
    mesh = plsc.VectorSubcoreMesh(core_axis_name="c", subcore_axis_name="s",
                                  num_cores=2, num_subcores=16)
    sc_fn = functools.partial(
        pl.kernel,
        out_type=jax.ShapeDtypeStruct((n * _K,), jnp.int32),
        mesh=mesh,
        scratch_types=[
            pltpu.VMEM((_SL, 128), jnp.float32),
            pltpu.VMEM((_SL, 128), jnp.float32),
            pltpu.VMEM((128 * _CAP,), jnp.int32),
            pltpu.VMEM((128 * _CAP,), jnp.float32),
            pltpu.VMEM((128,), jnp.int32),
            pltpu.VMEM((rows_w * _K,), jnp.int32),
            pltpu.VMEM((rows_w,), jnp.float32),
            pltpu.SemaphoreType.DMA,
            pltpu.SemaphoreType.DMA,
        ],
        compiler_params=pltpu.CompilerParams(use_tc_tiling_on_sc=True,
                                             needs_layout_passes=False),
    )(_sc_body)

    outs = []
    for bi in range(b):
        dist, thr = tc_fn(inputs[bi:bi + 1], inputs[bi:bi + 1])
        outs.append(sc_fn(dist, thr.reshape(n)))
    return jnp.stack(outs).reshape(b, n, _K)
